# baseline (speedup 1.0000x reference)
import dataclasses
import functools

import jax
import jax.numpy as jnp
from jax import lax
from jax.experimental import pallas as pl
from jax.experimental.pallas import tpu as pltpu
from jax.experimental.pallas import tpu_sc as plsc

NC = 2
NS = 16
NW = NC * NS
L = 16
C = 128


def _round_up(a, b):
    return (a + b - 1) // b * b


def _sc_compiler_params():
    cp = pltpu.CompilerParams()
    if "needs_layout_passes" in pltpu.CompilerParams.__dataclass_fields__:
        cp = dataclasses.replace(cp, needs_layout_passes=False)
    return cp


def _sc_degree(src_flat, np_pad):
    ew = src_flat.shape[1]
    mesh = plsc.VectorSubcoreMesh(core_axis_name="c", subcore_axis_name="s")

    @functools.partial(
        pl.kernel,
        out_type=jax.ShapeDtypeStruct((NW, np_pad), jnp.float32),
        mesh=mesh,
        scratch_types=[
            pltpu.VMEM((ew,), jnp.int32),
            pltpu.VMEM((np_pad,), jnp.float32),
            pltpu.SemaphoreType.DMA,
        ],
        compiler_params=_sc_compiler_params(),
    )
    def deg_kernel(src_hbm, out_hbm, idx_v, acc_v, sem):
        wid = lax.axis_index("s") * NC + lax.axis_index("c")
        pltpu.async_copy(src_hbm.at[wid], idx_v, sem).wait()
        zeros16 = jnp.zeros((L,), jnp.float32)

        @pl.loop(0, np_pad // L)
        def _(i):
            acc_v[pl.ds(i * L, L)] = zeros16

        ones16 = jnp.ones((L,), jnp.float32)

        @pl.loop(0, ew // L)
        def _(i):
            idx = idx_v[pl.ds(i * L, L)]
            plsc.addupdate_scatter(acc_v, [idx], ones16)

        pltpu.async_copy(acc_v, out_hbm.at[wid], sem).wait()

    return deg_kernel(src_flat)


def _sc_spmv(xs, srcp, dstp, zero_nd, k0, k1, nacc):
    np_pad, d = xs.shape
    rows_per = nacc // NS
    mesh = plsc.VectorSubcoreMesh(core_axis_name="c", subcore_axis_name="s")

    @functools.partial(
        pl.kernel,
        out_type=jax.ShapeDtypeStruct((NC, np_pad, d), jnp.float32),
        mesh=mesh,
        scratch_types=[
            pltpu.VMEM_SHARED((nacc, d), jnp.float32),
            pltpu.VMEM((k0, C), jnp.int32),
            pltpu.VMEM((C,), jnp.int32),
            pltpu.VMEM((C,), jnp.int32),
            pltpu.VMEM((C, d), jnp.float32),
            pltpu.VMEM((C, d), jnp.float32),
            pltpu.SemaphoreType.DMA,
            pltpu.SemaphoreType.DMA,
            pltpu.SemaphoreType.DMA,
            pltpu.SemaphoreType.DMA,
        ],
        compiler_params=_sc_compiler_params(),
    )
    def spmv_kernel(xs_hbm, srcp_hbm, dstp_hbm, zero_hbm, out_hbm,
                    acc_sh, src_v, didx0, didx1, rows0, rows1,
                    g0, g1, i0, i1):
        c = lax.axis_index("c")
        s = lax.axis_index("s")

        def run_edges(base, my_kc):
            half = my_kc // 2
            pltpu.sync_copy(srcp_hbm.at[s].at[pl.ds(base, my_kc)],
                            src_v.at[pl.ds(0, my_kc)])
            pltpu.async_copy(dstp_hbm.at[s].at[base], didx0, i0)
            pltpu.async_copy(dstp_hbm.at[s].at[base + 1], didx1, i1)
            pltpu.sync_copy(zero_hbm.at[pl.ds(s * rows_per, rows_per)],
                            acc_sh.at[pl.ds(s * rows_per, rows_per)])
            plsc.subcore_barrier()

            pltpu.async_copy(xs_hbm.at[src_v.at[0]], rows0, g0)

            @pl.loop(0, half)
            def _(t):
                k = t * 2
                pltpu.async_copy(xs_hbm.at[src_v.at[k + 1]], rows1, g1)
                pltpu.make_async_copy(xs_hbm.at[src_v.at[k]], rows0,
                                      g0).wait()
                pltpu.make_async_copy(dstp_hbm.at[s].at[base + k], didx0,
                                      i0).wait()
                pltpu.sync_copy(rows0, acc_sh.at[didx0], add=True)

                @pl.when(t < half - 1)
                def _():
                    pltpu.async_copy(dstp_hbm.at[s].at[base + k + 2],
                                     didx0, i0)
                    pltpu.async_copy(xs_hbm.at[src_v.at[k + 2]], rows0, g0)

                pltpu.make_async_copy(xs_hbm.at[src_v.at[k + 1]], rows1,
                                      g1).wait()
                pltpu.make_async_copy(dstp_hbm.at[s].at[base + k + 1],
                                      didx1, i1).wait()
                pltpu.sync_copy(rows1, acc_sh.at[didx1], add=True)

                @pl.when(t < half - 1)
                def _():
                    pltpu.async_copy(dstp_hbm.at[s].at[base + k + 3],
                                     didx1, i1)

        @pl.when(c == 0)
        def _():
            run_edges(0, k0)

        @pl.when(c == 1)
        def _():
            run_edges(k0, k1)

        plsc.subcore_barrier()
        pltpu.sync_copy(acc_sh.at[pl.ds(s * rows_per, rows_per)],
                        out_hbm.at[c].at[pl.ds(s * rows_per, rows_per)])

        @pl.when(s == NS - 1)
        def _():
            pltpu.sync_copy(zero_hbm.at[pl.ds(nacc, np_pad - nacc)],
                            out_hbm.at[c].at[pl.ds(nacc, np_pad - nacc)])

    return spmv_kernel(xs, srcp, dstp, zero_nd)


def _tc_prepare(deg_part, x_pad):
    np_pad, d = x_pad.shape
    r = np_pad // 8

    def body(deg_ref, x_ref, dinv_ref, xs_ref):
        deg = jnp.sum(deg_ref[...], axis=0)
        dinv = jnp.where(deg > 0, lax.rsqrt(deg), 0.0)[:, None]
        dinv_ref[...] = dinv
        xs_ref[...] = x_ref[...] * dinv

    return pl.pallas_call(
        body,
        grid=(np_pad // r,),
        in_specs=[pl.BlockSpec((NW, r), lambda i: (0, i)),
                  pl.BlockSpec((r, d), lambda i: (i, 0))],
        out_specs=[pl.BlockSpec((r, 1), lambda i: (i, 0)),
                   pl.BlockSpec((r, d), lambda i: (i, 0))],
        out_shape=[jax.ShapeDtypeStruct((np_pad, 1), jnp.float32),
                   jax.ShapeDtypeStruct((np_pad, d), jnp.float32)],
    )(deg_part, x_pad)


def _tc_combine(parts, dinv):
    np_pad, d = parts.shape[1], parts.shape[2]
    r = np_pad // 8

    def body(p_ref, dinv_ref, tx1_ref, xs2_ref):
        s = p_ref[0] + p_ref[1]
        dinv = dinv_ref[...]
        tx1 = -dinv * s
        tx1_ref[...] = tx1
        xs2_ref[...] = dinv * tx1

    return pl.pallas_call(
        body,
        grid=(np_pad // r,),
        in_specs=[pl.BlockSpec((NC, r, d), lambda i: (0, i, 0)),
                  pl.BlockSpec((r, 1), lambda i: (i, 0))],
        out_specs=[pl.BlockSpec((r, d), lambda i: (i, 0)),
                   pl.BlockSpec((r, d), lambda i: (i, 0))],
        out_shape=[jax.ShapeDtypeStruct((np_pad, d), jnp.float32),
                   jax.ShapeDtypeStruct((np_pad, d), jnp.float32)],
    )(parts, dinv)


def _tc_final(x_pad, tx1, qparts, dinv, w_cat, b_x, b_h, lin_w, lin_b, df):
    np_pad, d = x_pad.shape
    r = np_pad // 8

    def body(x_ref, t1_ref, q_ref, dinv_ref, w_ref, bx_ref, bh_ref,
             lw_ref, lb_ref, out_ref):
        xb = x_ref[...]
        t2 = -2.0 * dinv_ref[...] * (q_ref[0] + q_ref[1]) - xb
        cat = jnp.concatenate([xb, t1_ref[...], t2], axis=1)
        y = (jnp.dot(cat, w_ref[...], preferred_element_type=jnp.float32)
             + bx_ref[...] + bh_ref[...])
        z = jax.nn.sigmoid(y[:, :df])
        ht = jnp.tanh(y[:, df:])
        h = jnp.maximum((1.0 - z) * ht, 0.0)
        out_ref[...] = (jnp.dot(h, lw_ref[...],
                                preferred_element_type=jnp.float32)
                        + lb_ref[...])

    return pl.pallas_call(
        body,
        grid=(np_pad // r,),
        in_specs=[pl.BlockSpec((r, d), lambda i: (i, 0)),
                  pl.BlockSpec((r, d), lambda i: (i, 0)),
                  pl.BlockSpec((NC, r, d), lambda i: (0, i, 0)),
                  pl.BlockSpec((r, 1), lambda i: (i, 0)),
                  pl.BlockSpec((3 * d, 2 * df), lambda i: (0, 0)),
                  pl.BlockSpec((1, 2 * df), lambda i: (0, 0)),
                  pl.BlockSpec((1, 2 * df), lambda i: (0, 0)),
                  pl.BlockSpec((df, d), lambda i: (0, 0)),
                  pl.BlockSpec((1, d), lambda i: (0, 0))],
        out_specs=pl.BlockSpec((r, d), lambda i: (i, 0)),
        out_shape=jax.ShapeDtypeStruct((np_pad, d), jnp.float32),
    )(x_pad, tx1, qparts, dinv, w_cat, b_x, b_h, lin_w, lin_b)


def kernel(x, edge_index, Wz_x, bz_x, Wz_h, bz_h, Wr_x, br_x, Wr_h, br_h,
           Wh_x, bh_x, Wh_h, bh_h, lin_W, lin_b):
    n, d = x.shape
    df = Wz_x.shape[-1]
    e = edge_index.shape[1]
    np_pad = _round_up(n + 1, 2048)
    nacc = _round_up(n + 1, NS * 8)
    kc = _round_up(max(_round_up(e, NS * C) // (NS * C), 16), 8)
    k0 = min(_round_up(int(kc * 0.50), 8), kc - 8, 128)
    k1 = kc - k0
    ep = NS * kc * C

    src = edge_index[0].astype(jnp.int32)
    dst = edge_index[1].astype(jnp.int32)
    padv = jnp.full((ep - e,), n, jnp.int32)
    src_pad = jnp.concatenate([src, padv])
    dst_pad = jnp.concatenate([dst, padv])
    src_flat = src_pad.reshape(NW, ep // NW)
    srcp = src_pad.reshape(NS, kc, C)
    dstp = dst_pad.reshape(NS, kc, C)
    x_padded = jnp.concatenate([x, jnp.zeros((np_pad - n, d), x.dtype)],
                               axis=0)
    zero_nd = jnp.zeros((np_pad, d), jnp.float32)

    deg_part = _sc_degree(src_flat, np_pad)
    dinv, xs = _tc_prepare(deg_part, x_padded)
    p = _sc_spmv(xs, srcp, dstp, zero_nd, k0, k1, nacc)
    tx1, xs2 = _tc_combine(p, dinv)
    q = _sc_spmv(xs2, srcp, dstp, zero_nd, k0, k1, nacc)

    w_cat = jnp.concatenate(
        [jnp.concatenate([Wz_x[k], Wh_x[k]], axis=1) for k in range(3)],
        axis=0)
    b_x = jnp.concatenate([bz_x, bh_x]).reshape(1, 2 * df)
    b_h = jnp.concatenate([bz_h, bh_h]).reshape(1, 2 * df)
    out = _tc_final(x_padded, tx1, q, dinv, w_cat, b_x, b_h,
                    lin_W, lin_b.reshape(1, d), df)
    return out[:n]


# speedup vs baseline: 1.3425x; 1.3425x over previous
import dataclasses
import functools

import jax
import jax.numpy as jnp
from jax import lax
from jax.experimental import pallas as pl
from jax.experimental.pallas import tpu as pltpu
from jax.experimental.pallas import tpu_sc as plsc

NC = 2
NS = 16
NW = NC * NS
L = 16
C = 128


def _round_up(a, b):
    return (a + b - 1) // b * b


def _sc_compiler_params():
    cp = pltpu.CompilerParams()
    if "needs_layout_passes" in pltpu.CompilerParams.__dataclass_fields__:
        cp = dataclasses.replace(cp, needs_layout_passes=False)
    return cp


def _sc_degree(src_flat, np_pad):
    ew = src_flat.shape[1]
    mesh = plsc.VectorSubcoreMesh(core_axis_name="c", subcore_axis_name="s")

    @functools.partial(
        pl.kernel,
        out_type=jax.ShapeDtypeStruct((NW, np_pad), jnp.float32),
        mesh=mesh,
        scratch_types=[
            pltpu.VMEM((ew,), jnp.int32),
            pltpu.VMEM((np_pad,), jnp.float32),
            pltpu.SemaphoreType.DMA,
        ],
        compiler_params=_sc_compiler_params(),
    )
    def deg_kernel(src_hbm, out_hbm, idx_v, acc_v, sem):
        wid = lax.axis_index("s") * NC + lax.axis_index("c")
        pltpu.async_copy(src_hbm.at[wid], idx_v, sem).wait()
        zeros16 = jnp.zeros((L,), jnp.float32)

        @pl.loop(0, np_pad // L)
        def _(i):
            acc_v[pl.ds(i * L, L)] = zeros16

        ones16 = jnp.ones((L,), jnp.float32)

        @pl.loop(0, ew // L)
        def _(i):
            idx = idx_v[pl.ds(i * L, L)]
            plsc.addupdate_scatter(acc_v, [idx], ones16)

        pltpu.async_copy(acc_v, out_hbm.at[wid], sem).wait()

    return deg_kernel(src_flat)


def _sc_spmv(xs, srcp, dstp, zero_nd, k0, k1, nacc):
    np_pad, d = xs.shape
    rows_per = nacc // NS
    mesh = plsc.VectorSubcoreMesh(core_axis_name="c", subcore_axis_name="s")

    @functools.partial(
        pl.kernel,
        out_type=jax.ShapeDtypeStruct((NC, np_pad, d), jnp.float32),
        mesh=mesh,
        scratch_types=[
            pltpu.VMEM_SHARED((nacc, d), jnp.float32),
            pltpu.VMEM((k0, C), jnp.int32),
            pltpu.VMEM((C,), jnp.int32),
            pltpu.VMEM((C,), jnp.int32),
            pltpu.VMEM((C, d), jnp.float32),
            pltpu.VMEM((C, d), jnp.float32),
            pltpu.SemaphoreType.DMA,
            pltpu.SemaphoreType.DMA,
            pltpu.SemaphoreType.DMA,
            pltpu.SemaphoreType.DMA,
        ],
        compiler_params=_sc_compiler_params(),
    )
    def spmv_kernel(xs_hbm, srcp_hbm, dstp_hbm, zero_hbm, out_hbm,
                    acc_sh, src_v, didx0, didx1, rows0, rows1,
                    g0, g1, i0, i1):
        c = lax.axis_index("c")
        s = lax.axis_index("s")

        def run_edges(base, my_kc):
            half = my_kc // 2
            pltpu.sync_copy(srcp_hbm.at[s].at[pl.ds(base, my_kc)],
                            src_v.at[pl.ds(0, my_kc)])
            pltpu.async_copy(dstp_hbm.at[s].at[base], didx0, i0)
            pltpu.async_copy(dstp_hbm.at[s].at[base + 1], didx1, i1)
            pltpu.sync_copy(zero_hbm.at[pl.ds(s * rows_per, rows_per)],
                            acc_sh.at[pl.ds(s * rows_per, rows_per)])
            plsc.subcore_barrier()

            pltpu.async_copy(xs_hbm.at[src_v.at[0]], rows0, g0)

            @pl.loop(0, half)
            def _(t):
                k = t * 2
                pltpu.async_copy(xs_hbm.at[src_v.at[k + 1]], rows1, g1)
                pltpu.make_async_copy(xs_hbm.at[src_v.at[k]], rows0,
                                      g0).wait()
                pltpu.make_async_copy(dstp_hbm.at[s].at[base + k], didx0,
                                      i0).wait()
                pltpu.sync_copy(rows0, acc_sh.at[didx0], add=True)

                @pl.when(t < half - 1)
                def _():
                    pltpu.async_copy(dstp_hbm.at[s].at[base + k + 2],
                                     didx0, i0)
                    pltpu.async_copy(xs_hbm.at[src_v.at[k + 2]], rows0, g0)

                pltpu.make_async_copy(xs_hbm.at[src_v.at[k + 1]], rows1,
                                      g1).wait()
                pltpu.make_async_copy(dstp_hbm.at[s].at[base + k + 1],
                                      didx1, i1).wait()
                pltpu.sync_copy(rows1, acc_sh.at[didx1], add=True)

                @pl.when(t < half - 1)
                def _():
                    pltpu.async_copy(dstp_hbm.at[s].at[base + k + 3],
                                     didx1, i1)

        @pl.when(c == 0)
        def _():
            run_edges(0, k0)

        @pl.when(c == 1)
        def _():
            run_edges(k0, k1)

        plsc.subcore_barrier()
        pltpu.sync_copy(acc_sh.at[pl.ds(s * rows_per, rows_per)],
                        out_hbm.at[c].at[pl.ds(s * rows_per, rows_per)])

        if np_pad > nacc:
            @pl.when(s == NS - 1)
            def _():
                pltpu.sync_copy(zero_hbm.at[pl.ds(nacc, np_pad - nacc)],
                                out_hbm.at[c].at[pl.ds(nacc, np_pad - nacc)])

    return spmv_kernel(xs, srcp, dstp, zero_nd)


def _tc_prepare(deg_part, x_pad):
    np_pad, d = x_pad.shape
    r = np_pad // 8

    def body(deg_ref, x_ref, dinv_ref, xs_ref):
        deg = jnp.sum(deg_ref[...], axis=0)
        dinv = jnp.where(deg > 0, lax.rsqrt(deg), 0.0)[:, None]
        dinv_ref[...] = dinv
        xs_ref[...] = x_ref[...] * dinv

    return pl.pallas_call(
        body,
        grid=(np_pad // r,),
        in_specs=[pl.BlockSpec((NW, r), lambda i: (0, i)),
                  pl.BlockSpec((r, d), lambda i: (i, 0))],
        out_specs=[pl.BlockSpec((r, 1), lambda i: (i, 0)),
                   pl.BlockSpec((r, d), lambda i: (i, 0))],
        out_shape=[jax.ShapeDtypeStruct((np_pad, 1), jnp.float32),
                   jax.ShapeDtypeStruct((np_pad, d), jnp.float32)],
    )(deg_part, x_pad)


def _tc_combine(parts, dinv):
    np_pad, d = parts.shape[1], parts.shape[2]
    r = np_pad // 8

    def body(p_ref, dinv_ref, tx1_ref, xs2_ref):
        s = p_ref[0] + p_ref[1]
        dinv = dinv_ref[...]
        tx1 = -dinv * s
        tx1_ref[...] = tx1
        xs2_ref[...] = dinv * tx1

    return pl.pallas_call(
        body,
        grid=(np_pad // r,),
        in_specs=[pl.BlockSpec((NC, r, d), lambda i: (0, i, 0)),
                  pl.BlockSpec((r, 1), lambda i: (i, 0))],
        out_specs=[pl.BlockSpec((r, d), lambda i: (i, 0)),
                   pl.BlockSpec((r, d), lambda i: (i, 0))],
        out_shape=[jax.ShapeDtypeStruct((np_pad, d), jnp.float32),
                   jax.ShapeDtypeStruct((np_pad, d), jnp.float32)],
    )(parts, dinv)


def _tc_final(x_pad, tx1, qparts, dinv, w_cat, b_x, b_h, lin_w, lin_b, df):
    np_pad, d = x_pad.shape
    r = np_pad // 8

    def body(x_ref, t1_ref, q_ref, dinv_ref, w_ref, bx_ref, bh_ref,
             lw_ref, lb_ref, out_ref):
        xb = x_ref[...]
        t2 = -2.0 * dinv_ref[...] * (q_ref[0] + q_ref[1]) - xb
        cat = jnp.concatenate([xb, t1_ref[...], t2], axis=1)
        y = (jnp.dot(cat, w_ref[...], preferred_element_type=jnp.float32)
             + bx_ref[...] + bh_ref[...])
        z = jax.nn.sigmoid(y[:, :df])
        ht = jnp.tanh(y[:, df:])
        h = jnp.maximum((1.0 - z) * ht, 0.0)
        out_ref[...] = (jnp.dot(h, lw_ref[...],
                                preferred_element_type=jnp.float32)
                        + lb_ref[...])

    return pl.pallas_call(
        body,
        grid=(np_pad // r,),
        in_specs=[pl.BlockSpec((r, d), lambda i: (i, 0)),
                  pl.BlockSpec((r, d), lambda i: (i, 0)),
                  pl.BlockSpec((NC, r, d), lambda i: (0, i, 0)),
                  pl.BlockSpec((r, 1), lambda i: (i, 0)),
                  pl.BlockSpec((3 * d, 2 * df), lambda i: (0, 0)),
                  pl.BlockSpec((1, 2 * df), lambda i: (0, 0)),
                  pl.BlockSpec((1, 2 * df), lambda i: (0, 0)),
                  pl.BlockSpec((df, d), lambda i: (0, 0)),
                  pl.BlockSpec((1, d), lambda i: (0, 0))],
        out_specs=pl.BlockSpec((r, d), lambda i: (i, 0)),
        out_shape=jax.ShapeDtypeStruct((np_pad, d), jnp.float32),
    )(x_pad, tx1, qparts, dinv, w_cat, b_x, b_h, lin_w, lin_b)


def kernel(x, edge_index, Wz_x, bz_x, Wz_h, bz_h, Wr_x, br_x, Wr_h, br_h,
           Wh_x, bh_x, Wh_h, bh_h, lin_W, lin_b):
    n, d = x.shape
    df = Wz_x.shape[-1]
    e = edge_index.shape[1]
    np_pad = _round_up(n + 1, 2048)
    nacc = np_pad
    kc = _round_up(max(_round_up(e, NS * C) // (NS * C), 16), 8)
    k0 = min(_round_up(int(kc * 0.75), 8), kc - 8, 120)
    k1 = kc - k0
    ep = NS * kc * C

    src = edge_index[0].astype(jnp.int32)
    dst = edge_index[1].astype(jnp.int32)
    padv = jnp.full((ep - e,), n, jnp.int32)
    src_pad = jnp.concatenate([src, padv])
    dst_pad = jnp.concatenate([dst, padv])
    src_flat = src_pad.reshape(NW, ep // NW)
    srcp = src_pad.reshape(NS, kc, C)
    dstp = dst_pad.reshape(NS, kc, C)
    x_padded = jnp.concatenate([x, jnp.zeros((np_pad - n, d), x.dtype)],
                               axis=0)
    zero_nd = jnp.zeros((np_pad, d), jnp.float32)

    deg_part = _sc_degree(src_flat, np_pad)
    dinv, xs = _tc_prepare(deg_part, x_padded)
    p = _sc_spmv(xs, srcp, dstp, zero_nd, k0, k1, nacc)
    tx1, xs2 = _tc_combine(p, dinv)
    q = _sc_spmv(xs2, srcp, dstp, zero_nd, k0, k1, nacc)

    w_cat = jnp.concatenate(
        [jnp.concatenate([Wz_x[k], Wh_x[k]], axis=1) for k in range(3)],
        axis=0)
    b_x = jnp.concatenate([bz_x, bh_x]).reshape(1, 2 * df)
    b_h = jnp.concatenate([bz_h, bh_h]).reshape(1, 2 * df)
    out = _tc_final(x_padded, tx1, q, dinv, w_cat, b_x, b_h,
                    lin_W, lin_b.reshape(1, d), df)
    return out[:n]
